# Initial kernel scaffold; baseline (speedup 1.0000x reference)
import functools
import math

import jax
import jax.numpy as jnp
from jax import lax
from jax.experimental import pallas as pl
from jax.experimental.pallas import tpu as pltpu
from jax.experimental.pallas import tpu_sc as plsc

N_NODES = 10000
NP = 10240
NPT = NP // 16
E_EDGES = 320000
CHUNK = 128
E_PAD = ((E_EDGES + 16 * CHUNK - 1) // (16 * CHUNK)) * (16 * CHUNK)
EPT = E_PAD // 16
NCHUNK = EPT // CHUNK

_E = math.exp(1.0)
A_COEF = (_E + 1.0) * 1.0 / (_E - 1.0)
C_COEF = -1.0 / (_E - 1.0) + 0.5

_MESH = plsc.VectorSubcoreMesh(core_axis_name="c", subcore_axis_name="s")


@functools.partial(
    pl.kernel,
    out_type=jax.ShapeDtypeStruct((2 * NP,), jnp.float32),
    mesh=_MESH,
    scratch_types=[
        pltpu.VMEM((CHUNK,), jnp.int32),
        pltpu.VMEM((CHUNK, 16), jnp.float32),
        pltpu.VMEM((CHUNK, 16), jnp.float32),
        pltpu.VMEM((NPT,), jnp.float32),
        pltpu.VMEM_SHARED((NP, 16), jnp.float32),
    ],
)
def _sc_degrees(ef, out, ibuf, ones_b, dbuf, obuf, acc):
    c = lax.axis_index("c")
    s = lax.axis_index("s")
    one_hot = jnp.where(lax.iota(jnp.int32, 16) == 0, 1.0, 0.0)
    zeros = jnp.zeros((16,), jnp.float32)
    for j in range(CHUNK):
        ones_b[j, :] = one_hot
        dbuf[j, :] = zeros

    def zero_chunk(k, _):
        pltpu.sync_copy(dbuf, acc.at[pl.ds(s * NPT + k * CHUNK, CHUNK), :])
        return 0
    lax.fori_loop(0, NPT // CHUNK, zero_chunk, 0)
    plsc.subcore_barrier()

    def scat_chunk(j, _):
        off = c * E_PAD + s * EPT + j * CHUNK
        pltpu.sync_copy(ef.at[pl.ds(off, CHUNK)], ibuf)
        pltpu.sync_copy(ones_b, acc.at[ibuf], add=True)
        return 0
    lax.fori_loop(0, NCHUNK, scat_chunk, 0)
    plsc.subcore_barrier()

    lanes = lax.iota(jnp.int32, 16)
    col0 = jnp.zeros((16,), jnp.int32)

    def read_chunk(k, _):
        pltpu.sync_copy(acc.at[pl.ds(s * NPT + k * CHUNK, CHUNK), :], dbuf)
        for i in range(CHUNK // 16):
            v = plsc.load_gather(dbuf, [i * 16 + lanes, col0])
            obuf[pl.ds(k * CHUNK + i * 16, 16)] = v
        return 0
    lax.fori_loop(0, NPT // CHUNK, read_chunk, 0)
    pltpu.sync_copy(obuf, out.at[pl.ds(c * NP + s * NPT, NPT)])


def _make_segsum(width):
    @functools.partial(
        pl.kernel,
        out_type=[jax.ShapeDtypeStruct((NP, width), jnp.float32),
                  jax.ShapeDtypeStruct((NP, width), jnp.float32)],
        mesh=_MESH,
        scratch_types=[
            pltpu.VMEM((CHUNK,), jnp.int32),
            pltpu.VMEM((CHUNK,), jnp.int32),
            pltpu.VMEM((CHUNK, width), jnp.float32),
            pltpu.VMEM_SHARED((NP, width), jnp.float32),
            pltpu.SemaphoreType.DMA,
        ],
    )
    def segsum(y0, y1, ef, out0, out1, ibr, ibc, dbuf, acc, sem):
        c = lax.axis_index("c")
        s = lax.axis_index("s")

        def init(y_ref):
            def chunk(k, _):
                base = s * NPT + k * CHUNK
                pltpu.sync_copy(y_ref.at[pl.ds(base, CHUNK), :], dbuf)
                pltpu.sync_copy(dbuf, acc.at[pl.ds(base, CHUNK), :])
                return 0
            lax.fori_loop(0, NPT // CHUNK, chunk, 0)

        def scatter(y_ref):
            def chunk(j, _):
                off = s * EPT + j * CHUNK
                pltpu.sync_copy(ef.at[pl.ds(off, CHUNK)], ibr)
                pltpu.sync_copy(ef.at[pl.ds(E_PAD + off, CHUNK)], ibc)
                pltpu.async_copy(y_ref.at[ibr], dbuf, sem).wait()
                pltpu.sync_copy(dbuf, acc.at[ibc], add=True)
                return 0
            lax.fori_loop(0, NCHUNK, chunk, 0)

        def readout(out_ref):
            def chunk(k, _):
                base = s * NPT + k * CHUNK
                pltpu.sync_copy(acc.at[pl.ds(base, CHUNK), :], dbuf)
                pltpu.sync_copy(dbuf, out_ref.at[pl.ds(base, CHUNK), :])
                return 0
            lax.fori_loop(0, NPT // CHUNK, chunk, 0)

        @pl.when(c == 0)
        def _():
            init(y0)

        @pl.when(c == 1)
        def _():
            init(y1)

        plsc.subcore_barrier()

        @pl.when(c == 0)
        def _():
            scatter(y0)

        @pl.when(c == 1)
        def _():
            scatter(y1)

        plsc.subcore_barrier()

        @pl.when(c == 0)
        def _():
            readout(out0)

        @pl.when(c == 1)
        def _():
            readout(out1)

    return segsum


_segsum64 = _make_segsum(64)
_segsum128 = _make_segsum(128)


def _prep_body(cr_ref, cc_ref, x_ref, pv_ref, y0_ref, y1_ref, d1_ref, d2_ref):
    d1 = lax.rsqrt(cr_ref[...] + 1.0)
    d2 = lax.rsqrt(cc_ref[...] + 1.0)
    x = x_ref[...]
    m = jnp.where(pv_ref[...] > 0.0, A_COEF * x + C_COEF, x)
    y = d1 * m
    y0_ref[...] = y[:, :64]
    y1_ref[...] = y[:, 64:]
    d1_ref[...] = d1
    d2_ref[...] = d2


def _tc_prep(cnt_r, cnt_c, x_p, priv_f):
    rb = 1024
    grid = (NP // rb,)
    return pl.pallas_call(
        _prep_body,
        grid=grid,
        in_specs=[
            pl.BlockSpec((rb, 1), lambda i: (i, 0)),
            pl.BlockSpec((rb, 1), lambda i: (i, 0)),
            pl.BlockSpec((rb, 128), lambda i: (i, 0)),
            pl.BlockSpec((rb, 1), lambda i: (i, 0)),
        ],
        out_specs=[
            pl.BlockSpec((rb, 64), lambda i: (i, 0)),
            pl.BlockSpec((rb, 64), lambda i: (i, 0)),
            pl.BlockSpec((rb, 1), lambda i: (i, 0)),
            pl.BlockSpec((rb, 1), lambda i: (i, 0)),
        ],
        out_shape=[
            jax.ShapeDtypeStruct((NP, 64), jnp.float32),
            jax.ShapeDtypeStruct((NP, 64), jnp.float32),
            jax.ShapeDtypeStruct((NP, 1), jnp.float32),
            jax.ShapeDtypeStruct((NP, 1), jnp.float32),
        ],
    )(cnt_r, cnt_c, x_p, priv_f)


def _dense_body(a0_ref, a1_ref, d1_ref, d2_ref, w1_ref, b1_ref, wmu_ref,
                wlv_ref, omu_ref, olv_ref):
    h0 = d1_ref[...] * jnp.concatenate([a0_ref[...], a1_ref[...]], axis=1)
    h = jnp.maximum(
        jnp.dot(h0, w1_ref[...], preferred_element_type=jnp.float32)
        + b1_ref[...], 0.0)
    d2 = d2_ref[...]
    omu_ref[...] = d2 * jnp.dot(h, wmu_ref[...],
                                preferred_element_type=jnp.float32)
    olv_ref[...] = d2 * jnp.dot(h, wlv_ref[...],
                                preferred_element_type=jnp.float32)


def _tc_dense(a0, a1, dis1, dis2, W1, b1, Wmu, Wlv):
    rb = 1024
    grid = (NP // rb,)
    return pl.pallas_call(
        _dense_body,
        grid=grid,
        in_specs=[
            pl.BlockSpec((rb, 64), lambda i: (i, 0)),
            pl.BlockSpec((rb, 64), lambda i: (i, 0)),
            pl.BlockSpec((rb, 1), lambda i: (i, 0)),
            pl.BlockSpec((rb, 1), lambda i: (i, 0)),
            pl.BlockSpec((128, 256), lambda i: (0, 0)),
            pl.BlockSpec((1, 256), lambda i: (0, 0)),
            pl.BlockSpec((256, 128), lambda i: (0, 0)),
            pl.BlockSpec((256, 128), lambda i: (0, 0)),
        ],
        out_specs=[
            pl.BlockSpec((rb, 128), lambda i: (i, 0)),
            pl.BlockSpec((rb, 128), lambda i: (i, 0)),
        ],
        out_shape=[
            jax.ShapeDtypeStruct((NP, 128), jnp.float32),
            jax.ShapeDtypeStruct((NP, 128), jnp.float32),
        ],
    )(a0, a1, dis1, dis2, W1, b1, Wmu, Wlv)


def _final_body(amu_ref, alv_ref, d2_ref, bmu_ref, blv_ref, mu_ref, lv_ref):
    d2 = d2_ref[...]
    mu_ref[...] = d2 * amu_ref[...] + bmu_ref[...]
    lv_ref[...] = d2 * alv_ref[...] + blv_ref[...]


def _tc_final(amu, alv, dis2, bmu, blv):
    rb = 1024
    grid = (NP // rb,)
    return pl.pallas_call(
        _final_body,
        grid=grid,
        in_specs=[
            pl.BlockSpec((rb, 128), lambda i: (i, 0)),
            pl.BlockSpec((rb, 128), lambda i: (i, 0)),
            pl.BlockSpec((rb, 1), lambda i: (i, 0)),
            pl.BlockSpec((1, 128), lambda i: (0, 0)),
            pl.BlockSpec((1, 128), lambda i: (0, 0)),
        ],
        out_specs=[
            pl.BlockSpec((rb, 128), lambda i: (i, 0)),
            pl.BlockSpec((rb, 128), lambda i: (i, 0)),
        ],
        out_shape=[
            jax.ShapeDtypeStruct((NP, 128), jnp.float32),
            jax.ShapeDtypeStruct((NP, 128), jnp.float32),
        ],
    )(amu, alv, dis2, bmu, blv)


def kernel(x, W1, b1, Wmu, bmu, Wlv, blv, edge_index, priv_mask):
    n = x.shape[0]
    e = edge_index.shape[1]
    ef = jnp.pad(edge_index, ((0, 0), (0, E_PAD - e)),
                 constant_values=n).reshape(-1)
    x_p = jnp.pad(x, ((0, NP - n), (0, 0)))
    priv_f = jnp.pad(priv_mask.astype(jnp.float32), ((0, NP - n), (0, 0)))

    deg2 = _sc_degrees(ef)
    cnt_r = deg2[:NP].reshape(NP, 1)
    cnt_c = deg2[NP:].reshape(NP, 1)

    y0, y1, dis1, dis2 = _tc_prep(cnt_r, cnt_c, x_p, priv_f)
    a0, a1 = _segsum64(y0, y1, ef)
    wmu_a, wlv_a = _tc_dense(a0, a1, dis1, dis2, W1, b1.reshape(1, -1),
                             Wmu, Wlv)
    amu, alv = _segsum128(wmu_a, wlv_a, ef)
    mu_p, lv_p = _tc_final(amu, alv, dis2, bmu.reshape(1, -1),
                           blv.reshape(1, -1))
    return mu_p[:n], lv_p[:n]


# speedup vs baseline: 12.9686x; 12.9686x over previous
import functools
import math

import jax
import jax.numpy as jnp
from jax import lax
from jax.experimental import pallas as pl
from jax.experimental.pallas import tpu as pltpu
from jax.experimental.pallas import tpu_sc as plsc

N_NODES = 10000
NP = 10240
NPT = NP // 16
E_EDGES = 320000
CHUNK = 128
E_PAD = ((E_EDGES + 32 * CHUNK - 1) // (32 * CHUNK)) * (32 * CHUNK)
EPT = E_PAD // 16
NCHUNK = EPT // CHUNK
EPT2 = E_PAD // 32
NCHUNK2 = EPT2 // CHUNK

_E = math.exp(1.0)
A_COEF = (_E + 1.0) * 1.0 / (_E - 1.0)
C_COEF = -1.0 / (_E - 1.0) + 0.5

_MESH = plsc.VectorSubcoreMesh(core_axis_name="c", subcore_axis_name="s")


@functools.partial(
    pl.kernel,
    out_type=jax.ShapeDtypeStruct((2 * NP, 16), jnp.float32),
    mesh=_MESH,
    scratch_types=[
        pltpu.VMEM((CHUNK,), jnp.int32),
        pltpu.VMEM((CHUNK, 16), jnp.float32),
        pltpu.VMEM((CHUNK, 16), jnp.float32),
        pltpu.VMEM_SHARED((NP, 16), jnp.float32),
    ],
)
def _sc_degrees(ef, out, ibuf, ones_b, dbuf, acc):
    c = lax.axis_index("c")
    s = lax.axis_index("s")
    one_hot = jnp.where(lax.iota(jnp.int32, 16) == 0, 1.0, 0.0)
    zeros = jnp.zeros((16,), jnp.float32)
    for j in range(CHUNK):
        ones_b[j, :] = one_hot
        dbuf[j, :] = zeros

    def zero_chunk(k, _):
        pltpu.sync_copy(dbuf, acc.at[pl.ds(s * NPT + k * CHUNK, CHUNK), :])
        return 0
    lax.fori_loop(0, NPT // CHUNK, zero_chunk, 0)
    plsc.subcore_barrier()

    def scat_chunk(j, _):
        off = c * E_PAD + s * EPT + j * CHUNK
        pltpu.sync_copy(ef.at[pl.ds(off, CHUNK)], ibuf)
        pltpu.sync_copy(ones_b, acc.at[ibuf], add=True)
        return 0
    lax.fori_loop(0, NCHUNK, scat_chunk, 0)
    plsc.subcore_barrier()

    def read_chunk(k, _):
        base = s * NPT + k * CHUNK
        pltpu.sync_copy(acc.at[pl.ds(base, CHUNK), :], dbuf)
        pltpu.sync_copy(dbuf, out.at[pl.ds(c * NP + base, CHUNK), :])
        return 0
    lax.fori_loop(0, NPT // CHUNK, read_chunk, 0)


_SEG_SCRATCH = [
    pltpu.VMEM((CHUNK,), jnp.int32),
    pltpu.VMEM((CHUNK,), jnp.int32),
    pltpu.VMEM((CHUNK, 128), jnp.float32),
    pltpu.VMEM_SHARED((NP, 128), jnp.float32),
    pltpu.SemaphoreType.DMA,
]
_SEG_OUT = [jax.ShapeDtypeStruct((NP, 128), jnp.float32),
            jax.ShapeDtypeStruct((NP, 128), jnp.float32)]


def _seg_init(y_ref, acc, dbuf, s):
    def chunk(k, _):
        base = s * NPT + k * CHUNK
        pltpu.sync_copy(y_ref.at[pl.ds(base, CHUNK), :], dbuf)
        pltpu.sync_copy(dbuf, acc.at[pl.ds(base, CHUNK), :])
        return 0
    lax.fori_loop(0, NPT // CHUNK, chunk, 0)


def _seg_init_zero(y_ref, acc, dbuf, ibr, sem, s):
    for k in range(CHUNK // 16):
        ibr[pl.ds(k * 16, 16)] = jnp.full((16,), N_NODES, jnp.int32)
    pltpu.async_copy(y_ref.at[ibr], dbuf, sem).wait()

    def chunk(k, _):
        pltpu.sync_copy(dbuf, acc.at[pl.ds(s * NPT + k * CHUNK, CHUNK), :])
        return 0
    lax.fori_loop(0, NPT // CHUNK, chunk, 0)


def _seg_scatter(y_ref, ef, acc, dbuf, ibr, ibc, sem, base_e, nchunk):
    def chunk(j, _):
        off = base_e + j * CHUNK
        pltpu.sync_copy(ef.at[pl.ds(off, CHUNK)], ibr)
        pltpu.sync_copy(ef.at[pl.ds(E_PAD + off, CHUNK)], ibc)
        pltpu.async_copy(y_ref.at[ibr], dbuf, sem).wait()
        pltpu.sync_copy(dbuf, acc.at[ibc], add=True)
        return 0
    lax.fori_loop(0, nchunk, chunk, 0)


def _seg_readout(out_ref, acc, dbuf, s):
    def chunk(k, _):
        base = s * NPT + k * CHUNK
        pltpu.sync_copy(acc.at[pl.ds(base, CHUNK), :], dbuf)
        pltpu.sync_copy(dbuf, out_ref.at[pl.ds(base, CHUNK), :])
        return 0
    lax.fori_loop(0, NPT // CHUNK, chunk, 0)


@functools.partial(pl.kernel, out_type=_SEG_OUT, mesh=_MESH,
                   scratch_types=_SEG_SCRATCH)
def _segsum_edge_split(y, ef, out0, out1, ibr, ibc, dbuf, acc, sem):
    c = lax.axis_index("c")
    s = lax.axis_index("s")

    @pl.when(c == 0)
    def _():
        _seg_init(y, acc, dbuf, s)

    @pl.when(c == 1)
    def _():
        _seg_init_zero(y, acc, dbuf, ibr, sem, s)

    plsc.subcore_barrier()
    _seg_scatter(y, ef, acc, dbuf, ibr, ibc, sem,
                 (c * 16 + s) * EPT2, NCHUNK2)
    plsc.subcore_barrier()

    @pl.when(c == 0)
    def _():
        _seg_readout(out0, acc, dbuf, s)

    @pl.when(c == 1)
    def _():
        _seg_readout(out1, acc, dbuf, s)


@functools.partial(pl.kernel, out_type=_SEG_OUT, mesh=_MESH,
                   scratch_types=_SEG_SCRATCH)
def _segsum_pair(y0, y1, ef, out0, out1, ibr, ibc, dbuf, acc, sem):
    c = lax.axis_index("c")
    s = lax.axis_index("s")

    def run(y_ref, out_ref):
        _seg_init(y_ref, acc, dbuf, s)
        plsc.subcore_barrier()
        _seg_scatter(y_ref, ef, acc, dbuf, ibr, ibc, sem, s * EPT, NCHUNK)
        plsc.subcore_barrier()
        _seg_readout(out_ref, acc, dbuf, s)

    @pl.when(c == 0)
    def _():
        run(y0, out0)

    @pl.when(c == 1)
    def _():
        run(y1, out1)


def _prep_body(cr_ref, cc_ref, x_ref, pv_ref, y_ref, d1_ref, d2_ref):
    d1 = lax.rsqrt(cr_ref[...] + 1.0)
    d2 = lax.rsqrt(cc_ref[...] + 1.0)
    x = x_ref[...]
    m = jnp.where(pv_ref[...] > 0.0, A_COEF * x + C_COEF, x)
    y_ref[...] = d1 * m
    d1_ref[...] = d1
    d2_ref[...] = d2


def _tc_prep(cnt_r, cnt_c, x_p, priv_f):
    rb = 1024
    grid = (NP // rb,)
    return pl.pallas_call(
        _prep_body,
        grid=grid,
        in_specs=[
            pl.BlockSpec((rb, 1), lambda i: (i, 0)),
            pl.BlockSpec((rb, 1), lambda i: (i, 0)),
            pl.BlockSpec((rb, 128), lambda i: (i, 0)),
            pl.BlockSpec((rb, 1), lambda i: (i, 0)),
        ],
        out_specs=[
            pl.BlockSpec((rb, 128), lambda i: (i, 0)),
            pl.BlockSpec((rb, 1), lambda i: (i, 0)),
            pl.BlockSpec((rb, 1), lambda i: (i, 0)),
        ],
        out_shape=[
            jax.ShapeDtypeStruct((NP, 128), jnp.float32),
            jax.ShapeDtypeStruct((NP, 1), jnp.float32),
            jax.ShapeDtypeStruct((NP, 1), jnp.float32),
        ],
    )(cnt_r, cnt_c, x_p, priv_f)


def _dense_body(a0_ref, a1_ref, d1_ref, d2_ref, w1_ref, b1_ref, wmu_ref,
                wlv_ref, omu_ref, olv_ref):
    h0 = d1_ref[...] * (a0_ref[...] + a1_ref[...])
    h = jnp.maximum(
        jnp.dot(h0, w1_ref[...], preferred_element_type=jnp.float32)
        + b1_ref[...], 0.0)
    d2 = d2_ref[...]
    omu_ref[...] = d2 * jnp.dot(h, wmu_ref[...],
                                preferred_element_type=jnp.float32)
    olv_ref[...] = d2 * jnp.dot(h, wlv_ref[...],
                                preferred_element_type=jnp.float32)


def _tc_dense(a0, a1, dis1, dis2, W1, b1, Wmu, Wlv):
    rb = 1024
    grid = (NP // rb,)
    return pl.pallas_call(
        _dense_body,
        grid=grid,
        in_specs=[
            pl.BlockSpec((rb, 128), lambda i: (i, 0)),
            pl.BlockSpec((rb, 128), lambda i: (i, 0)),
            pl.BlockSpec((rb, 1), lambda i: (i, 0)),
            pl.BlockSpec((rb, 1), lambda i: (i, 0)),
            pl.BlockSpec((128, 256), lambda i: (0, 0)),
            pl.BlockSpec((1, 256), lambda i: (0, 0)),
            pl.BlockSpec((256, 128), lambda i: (0, 0)),
            pl.BlockSpec((256, 128), lambda i: (0, 0)),
        ],
        out_specs=[
            pl.BlockSpec((rb, 128), lambda i: (i, 0)),
            pl.BlockSpec((rb, 128), lambda i: (i, 0)),
        ],
        out_shape=[
            jax.ShapeDtypeStruct((NP, 128), jnp.float32),
            jax.ShapeDtypeStruct((NP, 128), jnp.float32),
        ],
    )(a0, a1, dis1, dis2, W1, b1, Wmu, Wlv)


def _final_body(amu_ref, alv_ref, d2_ref, bmu_ref, blv_ref, mu_ref, lv_ref):
    d2 = d2_ref[...]
    mu_ref[...] = d2 * amu_ref[...] + bmu_ref[...]
    lv_ref[...] = d2 * alv_ref[...] + blv_ref[...]


def _tc_final(amu, alv, dis2, bmu, blv):
    rb = 1024
    grid = (NP // rb,)
    return pl.pallas_call(
        _final_body,
        grid=grid,
        in_specs=[
            pl.BlockSpec((rb, 128), lambda i: (i, 0)),
            pl.BlockSpec((rb, 128), lambda i: (i, 0)),
            pl.BlockSpec((rb, 1), lambda i: (i, 0)),
            pl.BlockSpec((1, 128), lambda i: (0, 0)),
            pl.BlockSpec((1, 128), lambda i: (0, 0)),
        ],
        out_specs=[
            pl.BlockSpec((rb, 128), lambda i: (i, 0)),
            pl.BlockSpec((rb, 128), lambda i: (i, 0)),
        ],
        out_shape=[
            jax.ShapeDtypeStruct((NP, 128), jnp.float32),
            jax.ShapeDtypeStruct((NP, 128), jnp.float32),
        ],
    )(amu, alv, dis2, bmu, blv)


def kernel(x, W1, b1, Wmu, bmu, Wlv, blv, edge_index, priv_mask):
    n = x.shape[0]
    e = edge_index.shape[1]
    ef = jnp.pad(edge_index, ((0, 0), (0, E_PAD - e)),
                 constant_values=n).reshape(-1)
    x_p = jnp.pad(x, ((0, NP - n), (0, 0)))
    priv_f = jnp.pad(priv_mask.astype(jnp.float32), ((0, NP - n), (0, 0)))

    deg2 = _sc_degrees(ef)
    cnt_r = deg2[:NP, 0].reshape(NP, 1)
    cnt_c = deg2[NP:, 0].reshape(NP, 1)

    y, dis1, dis2 = _tc_prep(cnt_r, cnt_c, x_p, priv_f)
    a0, a1 = _segsum_edge_split(y, ef)
    wmu_a, wlv_a = _tc_dense(a0, a1, dis1, dis2, W1, b1.reshape(1, -1),
                             Wmu, Wlv)
    amu, alv = _segsum_pair(wmu_a, wlv_a, ef)
    mu_p, lv_p = _tc_final(amu, alv, dis2, bmu.reshape(1, -1),
                           blv.reshape(1, -1))
    return mu_p[:n], lv_p[:n]
